# Initial kernel scaffold; baseline (speedup 1.0000x reference)
import jax
import jax.numpy as jnp
from jax.experimental import pallas as pl


def kernel(pred, target):
    raise NotImplementedError("write your pallas kernel here")


# speedup vs baseline: 31.3432x; 31.3432x over previous
import jax
import jax.numpy as jnp
from jax.experimental import pallas as pl
from jax.experimental.pallas import tpu as pltpu

_NCLS = 16
_W = 512
_R = 64
_NJ = _W // _R


def _jaccard_kernel(pred_ref, tgt_ref, out_ref, acc_ref):
    j = pl.program_id(1)

    @pl.when(j == 0)
    def _init():
        acc_ref[...] = jnp.zeros_like(acc_ref)

    best = pred_ref[0, 0]
    lbl = jnp.zeros((_R, _W), jnp.int32)
    for c in range(1, _NCLS):
        xc = pred_ref[0, c]
        take = xc > best
        best = jnp.where(take, xc, best)
        lbl = jnp.where(take, jnp.int32(c), lbl)

    t = tgt_ref[0]
    w = jnp.where(lbl == t, jnp.int32(129), jnp.int32(1))

    for c in range(_NCLS):
        s = jnp.where(lbl == c, w, jnp.int32(0)) + jnp.where(
            t == c, jnp.int32(16384), jnp.int32(0))
        acc_ref[c] = acc_ref[c] + s

    @pl.when(j == _NJ - 1)
    def _finish():
        for c in range(_NCLS):
            a = acc_ref[c]
            out_ref[0, 0, c] = jnp.sum(a & 127)
            out_ref[0, 1, c] = jnp.sum((a >> 7) & 127)
            out_ref[0, 2, c] = jnp.sum(a >> 14)


def _histograms(pred, target, interpret=False):
    nb = pred.shape[0]
    return pl.pallas_call(
        _jaccard_kernel,
        grid=(nb, _NJ),
        in_specs=[
            pl.BlockSpec((1, _NCLS, _R, _W), lambda b, j: (b, 0, j, 0)),
            pl.BlockSpec((1, _R, _W), lambda b, j: (b, j, 0)),
        ],
        out_specs=pl.BlockSpec((1, 3, _NCLS), lambda b, j: (b, 0, 0),
                               memory_space=pltpu.SMEM),
        out_shape=jax.ShapeDtypeStruct((nb, 3, _NCLS), jnp.int32),
        scratch_shapes=[pltpu.VMEM((_NCLS, _R, _W), jnp.int32)],
        compiler_params=pltpu.CompilerParams(
            dimension_semantics=("parallel", "arbitrary")),
        interpret=interpret,
    )(pred, target)


def kernel(pred, target):
    hist = _histograms(pred, target)
    cnt = hist.sum(axis=0).astype(jnp.float32)
    cp, ci, ct = cnt[0], cnt[1], cnt[2]
    union = cp + ct - ci
    scores = jnp.where(union == 0, jnp.float32(1.0),
                       ci / jnp.where(union == 0, 1.0, union))
    return scores.mean()


# speedup vs baseline: 37.0774x; 1.1829x over previous
import jax
import jax.numpy as jnp
from jax.experimental import pallas as pl
from jax.experimental.pallas import tpu as pltpu

_NCLS = 16
_W = 512
_R = 64
_NJ = _W // _R

_M2 = 0x33333333
_M4 = 0x0F0F0F0F
_ARR = {0: 0, 2: 1, 1: 2, 3: 3}


def _jaccard_kernel(pred_ref, tgt_ref, out_ref, acc_ref):
    j = pl.program_id(1)

    @pl.when(j == 0)
    def _init():
        acc_ref[...] = jnp.zeros_like(acc_ref)

    best = pred_ref[0, 0]
    oh = jnp.full((_R, _W), 1, jnp.int32)
    for c in range(1, _NCLS):
        xc = pred_ref[0, c]
        take = xc > best
        best = jnp.where(take, xc, best)
        oh = jnp.where(take, jnp.int32(1 << (2 * c)), oh)

    t = tgt_ref[0]
    oh_t_f = pltpu.bitcast(((t << 1) + 127) << 23, jnp.float32)
    oh_t = jnp.round(oh_t_f).astype(jnp.int32)
    oh_i = jnp.where(oh == oh_t, oh, jnp.int32(0))

    for chain, z in enumerate((oh, oh_i, oh_t)):
        a = z[0:8] + z[8:16] + z[16:24]
        b = z[24:32] + z[32:40] + z[40:48]
        d = z[48:56] + z[56:64]
        e4 = (a & _M2) + (b & _M2) + (d & _M2)
        o4 = ((a >> 2) & _M2) + ((b >> 2) & _M2) + ((d >> 2) & _M2)
        acc_ref[chain, 0] = acc_ref[chain, 0] + (e4 & _M4)
        acc_ref[chain, 1] = acc_ref[chain, 1] + ((e4 >> 4) & _M4)
        acc_ref[chain, 2] = acc_ref[chain, 2] + (o4 & _M4)
        acc_ref[chain, 3] = acc_ref[chain, 3] + ((o4 >> 4) & _M4)

    @pl.when(j == _NJ - 1)
    def _finish():
        for chain in range(3):
            for c in range(_NCLS):
                cnt = (acc_ref[chain, _ARR[c & 3]] >> (8 * (c >> 2))) & 255
                out_ref[0, chain, c] = jnp.sum(cnt)


def _histograms(pred, target, interpret=False):
    nb = pred.shape[0]
    return pl.pallas_call(
        _jaccard_kernel,
        grid=(nb, _NJ),
        in_specs=[
            pl.BlockSpec((1, _NCLS, _R, _W), lambda b, j: (b, 0, j, 0)),
            pl.BlockSpec((1, _R, _W), lambda b, j: (b, j, 0)),
        ],
        out_specs=pl.BlockSpec((1, 3, _NCLS), lambda b, j: (b, 0, 0),
                               memory_space=pltpu.SMEM),
        out_shape=jax.ShapeDtypeStruct((nb, 3, _NCLS), jnp.int32),
        scratch_shapes=[pltpu.VMEM((3, 4, 8, _W), jnp.int32)],
        compiler_params=pltpu.CompilerParams(
            dimension_semantics=("parallel", "arbitrary")),
        interpret=interpret,
    )(pred, target)


def kernel(pred, target):
    hist = _histograms(pred, target)
    cnt = hist.sum(axis=0).astype(jnp.float32)
    cp, ci, ct = cnt[0], cnt[1], cnt[2]
    union = cp + ct - ci
    scores = jnp.where(union == 0, jnp.float32(1.0),
                       ci / jnp.where(union == 0, 1.0, union))
    return scores.mean()


# speedup vs baseline: 52.9883x; 1.4291x over previous
import jax
import jax.numpy as jnp
from jax.experimental import pallas as pl
from jax.experimental.pallas import tpu as pltpu

_NCLS = 16
_W = 512
_H = 512
_R = 64
_NS = _H // _R

_M2 = 0x33333333
_M4 = 0x0F0F0F0F
_ARR = {0: 0, 2: 1, 1: 2, 3: 3}


def _jaccard_kernel(pred_ref, tgt_ref, out_ref):
    accs = [[None] * 4 for _ in range(3)]

    for s in range(_NS):
        r0 = s * _R
        best = pred_ref[0, 0, r0:r0 + _R, :]
        oh = jnp.full((_R, _W), 1, jnp.int32)
        for c in range(1, _NCLS):
            xc = pred_ref[0, c, r0:r0 + _R, :]
            take = xc > best
            best = jnp.where(take, xc, best)
            oh = jnp.where(take, jnp.int32(1 << (2 * c)), oh)

        t = tgt_ref[0, r0:r0 + _R, :]
        oh_t_f = pltpu.bitcast(((t << 1) + 127) << 23, jnp.float32)
        oh_t = jnp.round(oh_t_f).astype(jnp.int32)
        oh_i = jnp.where(oh == oh_t, oh, jnp.int32(0))

        for chain, z in enumerate((oh, oh_i, oh_t)):
            a = z[0:8] + z[8:16] + z[16:24]
            b = z[24:32] + z[32:40] + z[40:48]
            d = z[48:56] + z[56:64]
            e4 = (a & _M2) + (b & _M2) + (d & _M2)
            o4 = ((a >> 2) & _M2) + ((b >> 2) & _M2) + ((d >> 2) & _M2)
            u = (e4 & _M4,
                 (e4 >> 4) & _M4,
                 o4 & _M4,
                 (o4 >> 4) & _M4)
            for k in range(4):
                accs[chain][k] = u[k] if s == 0 else accs[chain][k] + u[k]

    for chain in range(3):
        for c in range(_NCLS):
            cnt = (accs[chain][_ARR[c & 3]] >> (8 * (c >> 2))) & 255
            out_ref[0, chain, c] = jnp.sum(cnt)


def _histograms(pred, target, interpret=False):
    nb = pred.shape[0]
    return pl.pallas_call(
        _jaccard_kernel,
        grid=(nb,),
        in_specs=[
            pl.BlockSpec((1, _NCLS, _H, _W), lambda b: (b, 0, 0, 0)),
            pl.BlockSpec((1, _H, _W), lambda b: (b, 0, 0)),
        ],
        out_specs=pl.BlockSpec((1, 3, _NCLS), lambda b: (b, 0, 0),
                               memory_space=pltpu.SMEM),
        out_shape=jax.ShapeDtypeStruct((nb, 3, _NCLS), jnp.int32),
        compiler_params=pltpu.CompilerParams(
            dimension_semantics=("arbitrary",),
            vmem_limit_bytes=56 * 1024 * 1024),
        interpret=interpret,
    )(pred, target)


def kernel(pred, target):
    hist = _histograms(pred, target)
    cnt = hist.sum(axis=0).astype(jnp.float32)
    cp, ci, ct = cnt[0], cnt[1], cnt[2]
    union = cp + ct - ci
    scores = jnp.where(union == 0, jnp.float32(1.0),
                       ci / jnp.where(union == 0, 1.0, union))
    return scores.mean()


# speedup vs baseline: 55.8447x; 1.0539x over previous
import jax
import jax.numpy as jnp
from jax.experimental import pallas as pl
from jax.experimental.pallas import tpu as pltpu

_NCLS = 16
_W = 512
_H = 512
_R = 64
_NS = _H // _R

_M2 = 0x33333333
_M4 = 0x0F0F0F0F
_ARR = {0: 0, 2: 1, 1: 2, 3: 3}


def _make_kernel(nb):
    def _jaccard_kernel(pred_ref, tgt_ref, out_ref, hist_ref):
        b = pl.program_id(0)
        accs = [[None] * 4 for _ in range(3)]

        for s in range(_NS):
            r0 = s * _R
            best = pred_ref[0, 0, r0:r0 + _R, :]
            oh = jnp.full((_R, _W), 1, jnp.int32)
            for c in range(1, _NCLS):
                xc = pred_ref[0, c, r0:r0 + _R, :]
                take = xc > best
                best = jnp.where(take, xc, best)
                oh = jnp.where(take, jnp.int32(1 << (2 * c)), oh)

            t = tgt_ref[0, r0:r0 + _R, :]
            oh_t_f = pltpu.bitcast(((t << 1) + 127) << 23, jnp.float32)
            oh_t = jnp.round(oh_t_f).astype(jnp.int32)
            oh_i = jnp.where(oh == oh_t, oh, jnp.int32(0))

            for chain, z in enumerate((oh, oh_i, oh_t)):
                a = z[0:8] + z[8:16] + z[16:24]
                bb = z[24:32] + z[32:40] + z[40:48]
                d = z[48:56] + z[56:64]
                e4 = (a & _M2) + (bb & _M2) + (d & _M2)
                o4 = ((a >> 2) & _M2) + ((bb >> 2) & _M2) + ((d >> 2) & _M2)
                u = (e4 & _M4,
                     (e4 >> 4) & _M4,
                     o4 & _M4,
                     (o4 >> 4) & _M4)
                for k in range(4):
                    accs[chain][k] = u[k] if s == 0 else accs[chain][k] + u[k]

        for chain in range(3):
            for c in range(_NCLS):
                cnt = jnp.sum(
                    (accs[chain][_ARR[c & 3]] >> (8 * (c >> 2))) & 255)
                prev = jnp.where(b == 0, 0, hist_ref[chain, c])
                hist_ref[chain, c] = prev + cnt

        @pl.when(b == nb - 1)
        def _finish():
            tot = jnp.float32(0.0)
            for c in range(_NCLS):
                cp = hist_ref[0, c].astype(jnp.float32)
                ci = hist_ref[1, c].astype(jnp.float32)
                ct = hist_ref[2, c].astype(jnp.float32)
                union = cp + ct - ci
                tot = tot + jnp.where(
                    union == 0, jnp.float32(1.0),
                    ci / jnp.where(union == 0, jnp.float32(1.0), union))
            out_ref[0, 0] = tot * jnp.float32(1.0 / _NCLS)

    return _jaccard_kernel


def kernel(pred, target, interpret=False):
    nb = pred.shape[0]
    out = pl.pallas_call(
        _make_kernel(nb),
        grid=(nb,),
        in_specs=[
            pl.BlockSpec((1, _NCLS, _H, _W), lambda b: (b, 0, 0, 0)),
            pl.BlockSpec((1, _H, _W), lambda b: (b, 0, 0)),
        ],
        out_specs=pl.BlockSpec((1, 1), lambda b: (0, 0),
                               memory_space=pltpu.SMEM),
        out_shape=jax.ShapeDtypeStruct((1, 1), jnp.float32),
        scratch_shapes=[pltpu.SMEM((3, _NCLS), jnp.int32)],
        compiler_params=pltpu.CompilerParams(
            dimension_semantics=("arbitrary",),
            vmem_limit_bytes=56 * 1024 * 1024),
        interpret=interpret,
    )(pred, target)
    return jnp.reshape(out, ())
